# baseline (speedup 1.0000x reference)
import jax
import jax.numpy as jnp
from jax.experimental import pallas as pl

_N = 50000
_R = 3


def _rgcn_layer(h, src, dst, edge_type, W, root, bias):
    out = h @ root + bias
    for r in range(_R):
        m = (edge_type == r).astype(h.dtype)
        msg = (h[src] @ W[r]) * m[:, None]
        summed = jnp.zeros((_N, W.shape[2]), h.dtype).at[dst].add(msg)
        cnt = jnp.zeros((_N,), h.dtype).at[dst].add(m)
        out = out + summed / jnp.clip(cnt, 1.0)[:, None]
    return out


def _final_linear_kernel(sums_ref, cnts_ref, w_ref, b_ref, out_ref):
    g = sums_ref[...] / jnp.clip(cnts_ref[...], 1.0)
    out_ref[...] = g @ w_ref[...] + b_ref[...]


def kernel(x, edge_index, edge_type, batch, embed_table, W1, root1, b1, W2, root2, b2, lin_W, lin_b):
    src = edge_index[0]
    dst = edge_index[1]
    h = jnp.take(embed_table, x, axis=0)
    h = jax.nn.relu(_rgcn_layer(h, src, dst, edge_type, W1, root1, b1))
    h = jax.nn.relu(_rgcn_layer(h, src, dst, edge_type, W2, root2, b2))
    G = 128
    sums = jnp.zeros((G, h.shape[1]), h.dtype).at[batch].add(h)
    cnts = jnp.zeros((G, 1), h.dtype).at[batch].add(1.0)
    out = pl.pallas_call(
        _final_linear_kernel,
        out_shape=jax.ShapeDtypeStruct((G, lin_W.shape[1]), h.dtype),
    )(sums, cnts, lin_W, lin_b)
    return out


# speedup vs baseline: 1.9096x; 1.9096x over previous
import functools

import jax
import jax.numpy as jnp
from jax import lax
from jax.experimental import pallas as pl
from jax.experimental.pallas import tpu as pltpu
from jax.experimental.pallas import tpu_sc as plsc

N = 50000
NPAD = 50176
E = 800000
EPAD = 802816
D = 64
R = 3
G = 128
Q = 25088
DUMP = Q
GP = 136
PDUMP = G
BN = 512
NB = 98
RPT = NPAD // 32
EPT = EPAD // 16
SUP = 1024
NSUP = EPT // SUP
F32 = jnp.float32
I32 = jnp.int32

_mesh = plsc.VectorSubcoreMesh(core_axis_name="c", subcore_axis_name="s")
_SC_PARAMS = pltpu.CompilerParams(use_tc_tiling_on_sc=False,
                                  needs_layout_passes=False)


def _zero16():
    return jnp.zeros((16,), F32)


def _fill_zero_rows(ref, nrows):
    def body(j, carry):
        for c4 in range(4):
            ref[j, pl.ds(16 * c4, 16)] = _zero16()
        return carry
    lax.fori_loop(0, nrows, body, 0)


def _fill_zero_flat(ref, n):
    def body(k, carry):
        ref[pl.ds(k * 16, 16)] = _zero16()
        return carry
    lax.fori_loop(0, n // 16, body, 0)


def _embed_body(x_hbm, tab_hbm, h_hbm, x_v, rows_v, sem):
    cid = lax.axis_index("c")
    sid = lax.axis_index("s")
    wid = cid * 16 + sid
    base = wid * RPT
    pltpu.sync_copy(x_hbm.at[pl.ds(base, RPT)], x_v)

    def chunk(c, carry):
        off = c * 112
        pltpu.async_copy(tab_hbm.at[x_v.at[pl.ds(off, 112)]], rows_v, sem).wait()
        pltpu.sync_copy(rows_v, h_hbm.at[pl.ds(base + off, 112)])
        return carry
    lax.fori_loop(0, RPT // 112, chunk, 0)


def _embed_call(x_pad, embed_table):
    return pl.kernel(
        _embed_body,
        out_type=jax.ShapeDtypeStruct((NPAD, D), F32),
        mesh=_mesh,
        compiler_params=_SC_PARAMS,
        scratch_types=[
            pltpu.VMEM((RPT,), I32),
            pltpu.VMEM((112, D), F32),
            pltpu.SemaphoreType.DMA,
        ],
    )(x_pad, embed_table)


def _mm_body(h_ref, w_ref, b_ref, hroot_ref, t_ref):
    prod = lax.dot_general(h_ref[...], w_ref[...], (((1,), (0,)), ((), ())),
                           preferred_element_type=F32)
    hroot_ref[...] = prod[:, 0:64] + b_ref[...]
    t_ref[0] = prod[:, 64:128]
    t_ref[1] = prod[:, 128:192]
    t_ref[2] = prod[:, 192:256]


def _mm_call(h, wcat, b):
    return pl.pallas_call(
        _mm_body,
        grid=(NB,),
        in_specs=[
            pl.BlockSpec((BN, D), lambda i: (i, 0)),
            pl.BlockSpec((D, 4 * D), lambda i: (0, 0)),
            pl.BlockSpec((1, D), lambda i: (0, 0)),
        ],
        out_specs=[
            pl.BlockSpec((BN, D), lambda i: (i, 0)),
            pl.BlockSpec((3, BN, D), lambda i: (0, i, 0)),
        ],
        out_shape=[
            jax.ShapeDtypeStruct((NPAD, D), F32),
            jax.ShapeDtypeStruct((3, NPAD, D), F32),
        ],
    )(h, wcat, b)


def _nmm_body(hroot_ref, a0_ref, a1_ref, a2_ref, w_ref, b_ref, hroot2_ref, t_ref):
    hb = jnp.maximum(
        hroot_ref[...] + a0_ref[...] + a1_ref[...] + a2_ref[...], 0.0)
    prod = lax.dot_general(hb, w_ref[...], (((1,), (0,)), ((), ())),
                           preferred_element_type=F32)
    hroot2_ref[...] = prod[:, 0:64] + b_ref[...]
    t_ref[0] = prod[:, 64:128]
    t_ref[1] = prod[:, 128:192]
    t_ref[2] = prod[:, 192:256]


def _nmm_call(hroot, accn, wcat, b):
    acc_spec = lambda r: pl.BlockSpec((BN, D), lambda i, r=r: (r * NB + i, 0))
    return pl.pallas_call(
        _nmm_body,
        grid=(NB,),
        in_specs=[
            pl.BlockSpec((BN, D), lambda i: (i, 0)),
            acc_spec(0), acc_spec(1), acc_spec(2),
            pl.BlockSpec((D, 4 * D), lambda i: (0, 0)),
            pl.BlockSpec((1, D), lambda i: (0, 0)),
        ],
        out_specs=[
            pl.BlockSpec((BN, D), lambda i: (i, 0)),
            pl.BlockSpec((3, BN, D), lambda i: (0, i, 0)),
        ],
        out_shape=[
            jax.ShapeDtypeStruct((NPAD, D), F32),
            jax.ShapeDtypeStruct((3, NPAD, D), F32),
        ],
    )(hroot, accn, accn, accn, wcat, b)


def _edge_body(t_hbm, src_hbm, dst_hbm, typ_hbm, accn_hbm,
               src_v, dst_v, typ_v, gidx_v, lidx_v, rows_v,
               ones_v, zblk_v, zrow_v, cnt_v, inv_v, sem,
               acc_sh, cnt_sh):
    cid = lax.axis_index("c")
    sid = lax.axis_index("s")
    rbase = sid * (Q // 16)

    _fill_zero_rows(zblk_v, 112)
    _fill_zero_flat(zrow_v, Q // 16 + 16)
    for k in range(8):
        ones_v[pl.ds(16 * k, 16)] = jnp.ones((16,), F32)

    for p in range(3):
        qbase = (cid * 3 + p) * Q

        def zc(c14, carry):
            pltpu.sync_copy(zblk_v, acc_sh.at[pl.ds(rbase + c14 * 112, 112)])
            return carry
        lax.fori_loop(0, 14, zc, 0)
        pltpu.sync_copy(zrow_v.at[pl.ds(0, Q // 16)], cnt_sh.at[pl.ds(rbase, Q // 16)])

        @pl.when(sid == 0)
        def _():
            pltpu.sync_copy(zblk_v.at[pl.ds(0, 8)], acc_sh.at[pl.ds(Q, 8)])
            pltpu.sync_copy(zrow_v.at[pl.ds(0, 8)], cnt_sh.at[pl.ds(Q, 8)])

        plsc.subcore_barrier()

        def sup(s, carry):
            ebase = sid * EPT + s * SUP
            pltpu.sync_copy(src_hbm.at[pl.ds(ebase, SUP)], src_v)
            pltpu.sync_copy(dst_hbm.at[pl.ds(ebase, SUP)], dst_v)
            pltpu.sync_copy(typ_hbm.at[pl.ds(ebase, SUP)], typ_v)
            for j in range(SUP // 128):
                for i in range(8):
                    off = j * 128 + i * 16
                    s16 = src_v[pl.ds(off, 16)]
                    d16 = dst_v[pl.ds(off, 16)]
                    t16 = typ_v[pl.ds(off, 16)]
                    tn = t16 * NPAD
                    gidx_v[pl.ds(i * 16, 16)] = tn + s16
                    lq = tn + d16 - qbase
                    inb = (lq >= 0) & (lq < Q)
                    lidx_v[pl.ds(i * 16, 16)] = jnp.where(inb, lq, DUMP)
                pltpu.async_copy(t_hbm.at[gidx_v], rows_v, sem).wait()
                pltpu.sync_copy(rows_v, acc_sh.at[lidx_v], add=True)
                pltpu.sync_copy(ones_v, cnt_sh.at[lidx_v], add=True)
            return carry
        lax.fori_loop(0, NSUP, sup, 0)

        plsc.subcore_barrier()

        pltpu.sync_copy(cnt_sh.at[pl.ds(rbase, Q // 16)], cnt_v)

        def invb(k, carry):
            c16 = cnt_v[pl.ds(k * 16, 16)]
            inv_v[pl.ds(k * 16, 16)] = 1.0 / jnp.maximum(c16, 1.0)
            return carry
        lax.fori_loop(0, Q // 16 // 16, invb, 0)

        def nc(c14, carry):
            off = c14 * 112
            pltpu.sync_copy(acc_sh.at[pl.ds(rbase + off, 112)],
                            rows_v.at[pl.ds(0, 112)])

            def rowb(jr, carry2):
                ib = plsc.load_gather(inv_v, [jnp.zeros((16,), I32) + (off + jr)])
                for c4 in range(4):
                    sl = pl.ds(16 * c4, 16)
                    rows_v[jr, sl] = rows_v[jr, sl] * ib
                return carry2
            lax.fori_loop(0, 112, rowb, 0)
            pltpu.sync_copy(rows_v.at[pl.ds(0, 112)],
                            accn_hbm.at[pl.ds(qbase + rbase + off, 112)])
            return carry
        lax.fori_loop(0, 14, nc, 0)


def _edge_call(tflat, srcp, dstp, typp):
    return pl.kernel(
        _edge_body,
        out_type=jax.ShapeDtypeStruct((6 * Q, D), F32),
        mesh=_mesh,
        compiler_params=_SC_PARAMS,
        scratch_types=[
            pltpu.VMEM((SUP,), I32),
            pltpu.VMEM((SUP,), I32),
            pltpu.VMEM((SUP,), I32),
            pltpu.VMEM((128,), I32),
            pltpu.VMEM((128,), I32),
            pltpu.VMEM((128, D), F32),
            pltpu.VMEM((128,), F32),
            pltpu.VMEM((112, D), F32),
            pltpu.VMEM((Q // 16 + 16,), F32),
            pltpu.VMEM((Q // 16,), F32),
            pltpu.VMEM((Q // 16,), F32),
            pltpu.SemaphoreType.DMA,
            pltpu.VMEM_SHARED((Q + 8, D), F32),
            pltpu.VMEM_SHARED((Q + 8,), F32),
        ],
    )(tflat, srcp, dstp, typp)


def _pool_body(hroot_hbm, accn_hbm, batch_hbm, ps_hbm, pc_hbm,
               bidx_v, h_v, a_v, ones_v, zblk_v, pool_sh, pcnt_sh):
    cid = lax.axis_index("c")
    sid = lax.axis_index("s")
    wid = cid * 16 + sid
    nbase = wid * RPT

    _fill_zero_rows(zblk_v, 112)

    def ob(j, carry):
        for c4 in range(4):
            ones_v[j, pl.ds(16 * c4, 16)] = jnp.ones((16,), F32)
        return carry
    lax.fori_loop(0, 112, ob, 0)

    @pl.when(sid == 0)
    def _():
        pltpu.sync_copy(zblk_v, pool_sh.at[pl.ds(0, 112)])
        pltpu.sync_copy(zblk_v.at[pl.ds(0, GP - 112)], pool_sh.at[pl.ds(112, GP - 112)])
        pltpu.sync_copy(zblk_v, pcnt_sh.at[pl.ds(0, 112)])
        pltpu.sync_copy(zblk_v.at[pl.ds(0, GP - 112)], pcnt_sh.at[pl.ds(112, GP - 112)])

    plsc.subcore_barrier()

    def chunk(c, carry):
        off = nbase + c * 112
        pltpu.sync_copy(batch_hbm.at[pl.ds(off, 112)], bidx_v)
        pltpu.sync_copy(hroot_hbm.at[pl.ds(off, 112)], h_v)
        for r in range(3):
            pltpu.sync_copy(accn_hbm.at[pl.ds(r * NPAD + off, 112)], a_v)

            def addb(jr, carry2):
                for c4 in range(4):
                    sl = pl.ds(16 * c4, 16)
                    h_v[jr, sl] = h_v[jr, sl] + a_v[jr, sl]
                return carry2
            lax.fori_loop(0, 112, addb, 0)

        def relub(jr, carry2):
            for c4 in range(4):
                sl = pl.ds(16 * c4, 16)
                h_v[jr, sl] = jnp.maximum(h_v[jr, sl], 0.0)
            return carry2
        lax.fori_loop(0, 112, relub, 0)

        pltpu.sync_copy(h_v, pool_sh.at[bidx_v], add=True)
        pltpu.sync_copy(ones_v, pcnt_sh.at[bidx_v], add=True)
        return carry
    lax.fori_loop(0, RPT // 112, chunk, 0)

    plsc.subcore_barrier()

    @pl.when(sid == 0)
    def _():
        pltpu.sync_copy(pool_sh.at[pl.ds(0, 112)], h_v)
        pltpu.sync_copy(h_v, ps_hbm.at[pl.ds(cid * GP, 112)])
        pltpu.sync_copy(pool_sh.at[pl.ds(112, GP - 112)], h_v.at[pl.ds(0, GP - 112)])
        pltpu.sync_copy(h_v.at[pl.ds(0, GP - 112)], ps_hbm.at[pl.ds(cid * GP + 112, GP - 112)])
        pltpu.sync_copy(pcnt_sh.at[pl.ds(0, 112)], h_v)
        pltpu.sync_copy(h_v, pc_hbm.at[pl.ds(cid * GP, 112)])
        pltpu.sync_copy(pcnt_sh.at[pl.ds(112, GP - 112)], h_v.at[pl.ds(0, GP - 112)])
        pltpu.sync_copy(h_v.at[pl.ds(0, GP - 112)], pc_hbm.at[pl.ds(cid * GP + 112, GP - 112)])


def _pool_call(hroot2, accn2, batch_pad):
    return pl.kernel(
        _pool_body,
        out_type=[
            jax.ShapeDtypeStruct((2 * GP, D), F32),
            jax.ShapeDtypeStruct((2 * GP, D), F32),
        ],
        mesh=_mesh,
        compiler_params=_SC_PARAMS,
        scratch_types=[
            pltpu.VMEM((112,), I32),
            pltpu.VMEM((112, D), F32),
            pltpu.VMEM((112, D), F32),
            pltpu.VMEM((112, D), F32),
            pltpu.VMEM((112, D), F32),
            pltpu.VMEM_SHARED((GP, D), F32),
            pltpu.VMEM_SHARED((GP, D), F32),
        ],
    )(hroot2, accn2, batch_pad)


def _fin_body(ps_ref, pc_ref, w_ref, b_ref, out_ref):
    s = ps_ref[0:GP] + ps_ref[GP:2 * GP]
    c = pc_ref[0:GP] + pc_ref[GP:2 * GP]
    g = s[0:G] / jnp.maximum(c[0:G], 1.0)
    out_ref[...] = lax.dot_general(g, w_ref[...], (((1,), (0,)), ((), ())),
                                   preferred_element_type=F32) + b_ref[...]


def _fin_call(ps, pc, lin_W, lin_b):
    return pl.pallas_call(
        _fin_body,
        out_shape=jax.ShapeDtypeStruct((G, lin_W.shape[1]), F32),
    )(ps, pc, lin_W, lin_b.reshape(1, -1))


def kernel(x, edge_index, edge_type, batch, embed_table, W1, root1, b1,
           W2, root2, b2, lin_W, lin_b):
    x_pad = jnp.concatenate([x.astype(I32), jnp.zeros((NPAD - N,), I32)])
    srcp = jnp.concatenate([edge_index[0].astype(I32), jnp.zeros((EPAD - E,), I32)])
    dstp = jnp.concatenate([edge_index[1].astype(I32),
                            jnp.full((EPAD - E,), 4 * NPAD, I32)])
    typp = jnp.concatenate([edge_type.astype(I32), jnp.zeros((EPAD - E,), I32)])
    batch_pad = jnp.concatenate([batch.astype(I32), jnp.full((NPAD - N,), PDUMP, I32)])

    wcat1 = jnp.concatenate([root1, W1[0], W1[1], W1[2]], axis=1)
    wcat2 = jnp.concatenate([root2, W2[0], W2[1], W2[2]], axis=1)

    h = _embed_call(x_pad, embed_table)
    hroot1, t1 = _mm_call(h, wcat1, b1.reshape(1, -1))
    accn1 = _edge_call(t1.reshape(3 * NPAD, D), srcp, dstp, typp)
    hroot2, t2 = _nmm_call(hroot1, accn1, wcat2, b2.reshape(1, -1))
    accn2 = _edge_call(t2.reshape(3 * NPAD, D), srcp, dstp, typp)
    ps, pc = _pool_call(hroot2, accn2, batch_pad)
    return _fin_call(ps, pc, lin_W, lin_b)


# speedup vs baseline: 1.9336x; 1.0126x over previous
import functools

import jax
import jax.numpy as jnp
from jax import lax
from jax.experimental import pallas as pl
from jax.experimental.pallas import tpu as pltpu
from jax.experimental.pallas import tpu_sc as plsc

N = 50000
NPAD = 50176
E = 800000
EPAD = 802816
D = 64
R = 3
G = 128
Q = 25088
DUMP = Q
GP = 136
PDUMP = G
BN = 512
NB = 98
RPT = NPAD // 32
EPT = EPAD // 16
SUP = 512
NSUP = EPT // SUP
F32 = jnp.float32
I32 = jnp.int32

_mesh = plsc.VectorSubcoreMesh(core_axis_name="c", subcore_axis_name="s")
_SC_PARAMS = pltpu.CompilerParams(use_tc_tiling_on_sc=False,
                                  needs_layout_passes=False)


def _zero16():
    return jnp.zeros((16,), F32)


def _fill_zero_rows(ref, nrows):
    def body(j, carry):
        for c4 in range(4):
            ref[j, pl.ds(16 * c4, 16)] = _zero16()
        return carry
    lax.fori_loop(0, nrows, body, 0)


def _fill_zero_flat(ref, n):
    def body(k, carry):
        ref[pl.ds(k * 16, 16)] = _zero16()
        return carry
    lax.fori_loop(0, n // 16, body, 0)


def _embed_body(x_hbm, tab_hbm, h_hbm, x_v, rows_v, sem):
    cid = lax.axis_index("c")
    sid = lax.axis_index("s")
    wid = cid * 16 + sid
    base = wid * RPT
    pltpu.sync_copy(x_hbm.at[pl.ds(base, RPT)], x_v)

    def chunk(c, carry):
        off = c * 112
        pltpu.async_copy(tab_hbm.at[x_v.at[pl.ds(off, 112)]], rows_v, sem).wait()
        pltpu.sync_copy(rows_v, h_hbm.at[pl.ds(base + off, 112)])
        return carry
    lax.fori_loop(0, RPT // 112, chunk, 0)


def _embed_call(x_pad, embed_table):
    return pl.kernel(
        _embed_body,
        out_type=jax.ShapeDtypeStruct((NPAD, D), F32),
        mesh=_mesh,
        compiler_params=_SC_PARAMS,
        scratch_types=[
            pltpu.VMEM((RPT,), I32),
            pltpu.VMEM((112, D), F32),
            pltpu.SemaphoreType.DMA,
        ],
    )(x_pad, embed_table)


def _mm_body(h_ref, w_ref, b_ref, hroot_ref, t_ref):
    prod = lax.dot_general(h_ref[...], w_ref[...], (((1,), (0,)), ((), ())),
                           preferred_element_type=F32)
    hroot_ref[...] = prod[:, 0:64] + b_ref[...]
    t_ref[0] = prod[:, 64:128]
    t_ref[1] = prod[:, 128:192]
    t_ref[2] = prod[:, 192:256]


def _mm_call(h, wcat, b):
    return pl.pallas_call(
        _mm_body,
        grid=(NB,),
        in_specs=[
            pl.BlockSpec((BN, D), lambda i: (i, 0)),
            pl.BlockSpec((D, 4 * D), lambda i: (0, 0)),
            pl.BlockSpec((1, D), lambda i: (0, 0)),
        ],
        out_specs=[
            pl.BlockSpec((BN, D), lambda i: (i, 0)),
            pl.BlockSpec((3, BN, D), lambda i: (0, i, 0)),
        ],
        out_shape=[
            jax.ShapeDtypeStruct((NPAD, D), F32),
            jax.ShapeDtypeStruct((3, NPAD, D), F32),
        ],
    )(h, wcat, b)


def _nmm_body(hroot_ref, a0_ref, a1_ref, a2_ref, w_ref, b_ref, hroot2_ref, t_ref):
    hb = jnp.maximum(
        hroot_ref[...] + a0_ref[...] + a1_ref[...] + a2_ref[...], 0.0)
    prod = lax.dot_general(hb, w_ref[...], (((1,), (0,)), ((), ())),
                           preferred_element_type=F32)
    hroot2_ref[...] = prod[:, 0:64] + b_ref[...]
    t_ref[0] = prod[:, 64:128]
    t_ref[1] = prod[:, 128:192]
    t_ref[2] = prod[:, 192:256]


def _nmm_call(hroot, accn, wcat, b):
    acc_spec = lambda r: pl.BlockSpec((BN, D), lambda i, r=r: (r * NB + i, 0))
    return pl.pallas_call(
        _nmm_body,
        grid=(NB,),
        in_specs=[
            pl.BlockSpec((BN, D), lambda i: (i, 0)),
            acc_spec(0), acc_spec(1), acc_spec(2),
            pl.BlockSpec((D, 4 * D), lambda i: (0, 0)),
            pl.BlockSpec((1, D), lambda i: (0, 0)),
        ],
        out_specs=[
            pl.BlockSpec((BN, D), lambda i: (i, 0)),
            pl.BlockSpec((3, BN, D), lambda i: (0, i, 0)),
        ],
        out_shape=[
            jax.ShapeDtypeStruct((NPAD, D), F32),
            jax.ShapeDtypeStruct((3, NPAD, D), F32),
        ],
    )(hroot, accn, accn, accn, wcat, b)


CHUNK = 128


def _edge_body(t_hbm, src_hbm, dst_hbm, typ_hbm, accn_hbm,
               src_v, dst_v, typ_v, gidx_a, lidx_a, gidx_b, lidx_b,
               rows_a, rows_b, ones_v, zblk_v, inv_v,
               sem_a, sem_b, acc_sh, cnt_sh):
    cid = lax.axis_index("c")
    sid = lax.axis_index("s")
    rbase = sid * (Q // 16)

    _fill_zero_rows(zblk_v, 32)
    for k in range(CHUNK // 16):
        ones_v[pl.ds(16 * k, 16)] = jnp.ones((16,), F32)

    def stage(s):
        ebase = sid * EPT + s * SUP
        pltpu.sync_copy(src_hbm.at[pl.ds(ebase, SUP)], src_v)
        pltpu.sync_copy(dst_hbm.at[pl.ds(ebase, SUP)], dst_v)
        pltpu.sync_copy(typ_hbm.at[pl.ds(ebase, SUP)], typ_v)

    for p in range(3):
        qbase = (cid * 3 + p) * Q

        def cidx(off, gidx, lidx):
            for i in range(CHUNK // 16):
                o = off + i * 16
                s16 = src_v[pl.ds(o, 16)]
                d16 = dst_v[pl.ds(o, 16)]
                t16 = typ_v[pl.ds(o, 16)]
                tn = t16 * NPAD
                gidx[pl.ds(i * 16, 16)] = tn + s16
                lq = tn + d16 - qbase
                inb = (lq >= 0) & (lq < Q)
                lidx[pl.ds(i * 16, 16)] = jnp.where(inb, lq, DUMP)

        def scat(rows, lidx):
            pltpu.sync_copy(rows, acc_sh.at[lidx], add=True)
            pltpu.sync_copy(ones_v, cnt_sh.at[lidx], add=True)

        _fill_zero_flat(inv_v, Q // 16)

        def zc(c, carry):
            pltpu.sync_copy(zblk_v, acc_sh.at[pl.ds(rbase + c * 32, 32)])
            return carry
        lax.fori_loop(0, Q // 16 // 32, zc, 0)
        pltpu.sync_copy(inv_v, cnt_sh.at[pl.ds(rbase, Q // 16)])

        @pl.when(sid == 0)
        def _():
            pltpu.sync_copy(zblk_v.at[pl.ds(0, 8)], acc_sh.at[pl.ds(Q, 8)])
            pltpu.sync_copy(inv_v.at[pl.ds(0, 8)], cnt_sh.at[pl.ds(Q, 8)])

        plsc.subcore_barrier()

        stage(0)
        cidx(0, gidx_a, lidx_a)
        pltpu.async_copy(t_hbm.at[gidx_a], rows_a, sem_a)

        def sup(s, carry):
            cidx(CHUNK, gidx_b, lidx_b)
            pltpu.async_copy(t_hbm.at[gidx_b], rows_b, sem_b)
            pltpu.make_async_copy(t_hbm.at[gidx_a], rows_a, sem_a).wait()
            scat(rows_a, lidx_a)

            cidx(2 * CHUNK, gidx_a, lidx_a)
            pltpu.async_copy(t_hbm.at[gidx_a], rows_a, sem_a)
            pltpu.make_async_copy(t_hbm.at[gidx_b], rows_b, sem_b).wait()
            scat(rows_b, lidx_b)

            cidx(3 * CHUNK, gidx_b, lidx_b)
            pltpu.async_copy(t_hbm.at[gidx_b], rows_b, sem_b)
            pltpu.make_async_copy(t_hbm.at[gidx_a], rows_a, sem_a).wait()
            scat(rows_a, lidx_a)

            @pl.when(s < NSUP - 1)
            def _():
                stage(s + 1)
                cidx(0, gidx_a, lidx_a)
                pltpu.async_copy(t_hbm.at[gidx_a], rows_a, sem_a)

            pltpu.make_async_copy(t_hbm.at[gidx_b], rows_b, sem_b).wait()
            scat(rows_b, lidx_b)
            return carry
        lax.fori_loop(0, NSUP, sup, 0)

        plsc.subcore_barrier()

        pltpu.sync_copy(cnt_sh.at[pl.ds(rbase, Q // 16)], inv_v)

        def invb(k, carry):
            c16 = inv_v[pl.ds(k * 16, 16)]
            inv_v[pl.ds(k * 16, 16)] = 1.0 / jnp.maximum(c16, 1.0)
            return carry
        lax.fori_loop(0, Q // 16 // 16, invb, 0)

        def nc(c, carry):
            off = c * 112
            pltpu.sync_copy(acc_sh.at[pl.ds(rbase + off, 112)],
                            rows_a.at[pl.ds(0, 112)])

            def rowb(jr, carry2):
                ib = plsc.load_gather(inv_v, [jnp.zeros((16,), I32) + (off + jr)])
                for c4 in range(4):
                    sl = pl.ds(16 * c4, 16)
                    rows_a[jr, sl] = rows_a[jr, sl] * ib
                return carry2
            lax.fori_loop(0, 112, rowb, 0)
            pltpu.sync_copy(rows_a.at[pl.ds(0, 112)],
                            accn_hbm.at[pl.ds(qbase + rbase + off, 112)])
            return carry
        lax.fori_loop(0, 14, nc, 0)


def _edge_call(tflat, srcp, dstp, typp):
    return pl.kernel(
        _edge_body,
        out_type=jax.ShapeDtypeStruct((6 * Q, D), F32),
        mesh=_mesh,
        compiler_params=_SC_PARAMS,
        scratch_types=[
            pltpu.VMEM((SUP,), I32),
            pltpu.VMEM((SUP,), I32),
            pltpu.VMEM((SUP,), I32),
            pltpu.VMEM((CHUNK,), I32),
            pltpu.VMEM((CHUNK,), I32),
            pltpu.VMEM((CHUNK,), I32),
            pltpu.VMEM((CHUNK,), I32),
            pltpu.VMEM((CHUNK, D), F32),
            pltpu.VMEM((CHUNK, D), F32),
            pltpu.VMEM((CHUNK,), F32),
            pltpu.VMEM((32, D), F32),
            pltpu.VMEM((Q // 16,), F32),
            pltpu.SemaphoreType.DMA,
            pltpu.SemaphoreType.DMA,
            pltpu.VMEM_SHARED((Q + 8, D), F32),
            pltpu.VMEM_SHARED((Q + 8,), F32),
        ],
    )(tflat, srcp, dstp, typp)


def _pool_body(hroot_hbm, accn_hbm, batch_hbm, ps_hbm, pc_hbm,
               bidx_v, h_v, a_v, ones_v, zblk_v, pool_sh, pcnt_sh):
    cid = lax.axis_index("c")
    sid = lax.axis_index("s")
    wid = cid * 16 + sid
    nbase = wid * RPT

    _fill_zero_rows(zblk_v, 112)

    def ob(j, carry):
        for c4 in range(4):
            ones_v[j, pl.ds(16 * c4, 16)] = jnp.ones((16,), F32)
        return carry
    lax.fori_loop(0, 112, ob, 0)

    @pl.when(sid == 0)
    def _():
        pltpu.sync_copy(zblk_v, pool_sh.at[pl.ds(0, 112)])
        pltpu.sync_copy(zblk_v.at[pl.ds(0, GP - 112)], pool_sh.at[pl.ds(112, GP - 112)])
        pltpu.sync_copy(zblk_v, pcnt_sh.at[pl.ds(0, 112)])
        pltpu.sync_copy(zblk_v.at[pl.ds(0, GP - 112)], pcnt_sh.at[pl.ds(112, GP - 112)])

    plsc.subcore_barrier()

    def chunk(c, carry):
        off = nbase + c * 112
        pltpu.sync_copy(batch_hbm.at[pl.ds(off, 112)], bidx_v)
        pltpu.sync_copy(hroot_hbm.at[pl.ds(off, 112)], h_v)
        for r in range(3):
            pltpu.sync_copy(accn_hbm.at[pl.ds(r * NPAD + off, 112)], a_v)

            def addb(jr, carry2):
                for c4 in range(4):
                    sl = pl.ds(16 * c4, 16)
                    h_v[jr, sl] = h_v[jr, sl] + a_v[jr, sl]
                return carry2
            lax.fori_loop(0, 112, addb, 0)

        def relub(jr, carry2):
            for c4 in range(4):
                sl = pl.ds(16 * c4, 16)
                h_v[jr, sl] = jnp.maximum(h_v[jr, sl], 0.0)
            return carry2
        lax.fori_loop(0, 112, relub, 0)

        pltpu.sync_copy(h_v, pool_sh.at[bidx_v], add=True)
        pltpu.sync_copy(ones_v, pcnt_sh.at[bidx_v], add=True)
        return carry
    lax.fori_loop(0, RPT // 112, chunk, 0)

    plsc.subcore_barrier()

    @pl.when(sid == 0)
    def _():
        pltpu.sync_copy(pool_sh.at[pl.ds(0, 112)], h_v)
        pltpu.sync_copy(h_v, ps_hbm.at[pl.ds(cid * GP, 112)])
        pltpu.sync_copy(pool_sh.at[pl.ds(112, GP - 112)], h_v.at[pl.ds(0, GP - 112)])
        pltpu.sync_copy(h_v.at[pl.ds(0, GP - 112)], ps_hbm.at[pl.ds(cid * GP + 112, GP - 112)])
        pltpu.sync_copy(pcnt_sh.at[pl.ds(0, 112)], h_v)
        pltpu.sync_copy(h_v, pc_hbm.at[pl.ds(cid * GP, 112)])
        pltpu.sync_copy(pcnt_sh.at[pl.ds(112, GP - 112)], h_v.at[pl.ds(0, GP - 112)])
        pltpu.sync_copy(h_v.at[pl.ds(0, GP - 112)], pc_hbm.at[pl.ds(cid * GP + 112, GP - 112)])


def _pool_call(hroot2, accn2, batch_pad):
    return pl.kernel(
        _pool_body,
        out_type=[
            jax.ShapeDtypeStruct((2 * GP, D), F32),
            jax.ShapeDtypeStruct((2 * GP, D), F32),
        ],
        mesh=_mesh,
        compiler_params=_SC_PARAMS,
        scratch_types=[
            pltpu.VMEM((112,), I32),
            pltpu.VMEM((112, D), F32),
            pltpu.VMEM((112, D), F32),
            pltpu.VMEM((112, D), F32),
            pltpu.VMEM((112, D), F32),
            pltpu.VMEM_SHARED((GP, D), F32),
            pltpu.VMEM_SHARED((GP, D), F32),
        ],
    )(hroot2, accn2, batch_pad)


def _fin_body(ps_ref, pc_ref, w_ref, b_ref, out_ref):
    s = ps_ref[0:GP] + ps_ref[GP:2 * GP]
    c = pc_ref[0:GP] + pc_ref[GP:2 * GP]
    g = s[0:G] / jnp.maximum(c[0:G], 1.0)
    out_ref[...] = lax.dot_general(g, w_ref[...], (((1,), (0,)), ((), ())),
                                   preferred_element_type=F32) + b_ref[...]


def _fin_call(ps, pc, lin_W, lin_b):
    return pl.pallas_call(
        _fin_body,
        out_shape=jax.ShapeDtypeStruct((G, lin_W.shape[1]), F32),
    )(ps, pc, lin_W, lin_b.reshape(1, -1))


def kernel(x, edge_index, edge_type, batch, embed_table, W1, root1, b1,
           W2, root2, b2, lin_W, lin_b):
    x_pad = jnp.concatenate([x.astype(I32), jnp.zeros((NPAD - N,), I32)])
    srcp = jnp.concatenate([edge_index[0].astype(I32), jnp.zeros((EPAD - E,), I32)])
    dstp = jnp.concatenate([edge_index[1].astype(I32),
                            jnp.full((EPAD - E,), 4 * NPAD, I32)])
    typp = jnp.concatenate([edge_type.astype(I32), jnp.zeros((EPAD - E,), I32)])
    batch_pad = jnp.concatenate([batch.astype(I32), jnp.full((NPAD - N,), PDUMP, I32)])

    wcat1 = jnp.concatenate([root1, W1[0], W1[1], W1[2]], axis=1)
    wcat2 = jnp.concatenate([root2, W2[0], W2[1], W2[2]], axis=1)

    h = _embed_call(x_pad, embed_table)
    hroot1, t1 = _mm_call(h, wcat1, b1.reshape(1, -1))
    accn1 = _edge_call(t1.reshape(3 * NPAD, D), srcp, dstp, typp)
    hroot2, t2 = _nmm_call(hroot1, accn1, wcat2, b2.reshape(1, -1))
    accn2 = _edge_call(t2.reshape(3 * NPAD, D), srcp, dstp, typp)
    ps, pc = _pool_call(hroot2, accn2, batch_pad)
    return _fin_call(ps, pc, lin_W, lin_b)
